# baseline (speedup 1.0000x reference)
import functools

import jax
import jax.numpy as jnp
from jax import lax
from jax.experimental import pallas as pl
from jax.experimental.pallas import tpu as pltpu
from jax.experimental.pallas import tpu_sc as plsc

N_ROWS = 16384
N_COLS = 1000
N_SUB = 100

NW = 32
ROWS_PER_W = N_ROWS // NW
CHUNK_ROWS = 64
N_CHUNKS = ROWS_PER_W // CHUNK_ROWS
GROUPS_PER_CHUNK = CHUNK_ROWS // 4
N_VECS = (4 * N_SUB) // 16


def _sc_gather(x_flat, idx):
    mesh = plsc.VectorSubcoreMesh(core_axis_name="c", subcore_axis_name="s")

    @functools.partial(
        pl.kernel,
        mesh=mesh,
        out_type=jax.ShapeDtypeStruct((N_ROWS * N_SUB,), jnp.float32),
        compiler_params=pltpu.CompilerParams(needs_layout_passes=False),
        scratch_types=[
            pltpu.VMEM((N_SUB,), jnp.int32),
            pltpu.VMEM((4 * N_SUB,), jnp.int32),
            pltpu.VMEM((CHUNK_ROWS * N_COLS,), jnp.float32),
            pltpu.VMEM((CHUNK_ROWS * N_SUB,), jnp.float32),
        ],
    )
    def k(x_hbm, idx_hbm, out_hbm, idx_v, idx4_v, xbuf, obuf):
        wid = lax.axis_index("s") * 2 + lax.axis_index("c")
        base_row = wid * ROWS_PER_W

        pltpu.sync_copy(idx_hbm, idx_v)
        for v in range(N_VECS):
            kvec = lax.iota(jnp.int32, 16) + (v * 16)
            kmod = lax.rem(kvec, N_SUB)
            kdiv = lax.div(kvec, N_SUB)
            cols = plsc.load_gather(idx_v, [kmod])
            idx4_v[pl.ds(v * 16, 16)] = cols + kdiv * N_COLS

        def chunk_body(c, carry):
            row0 = base_row + c * CHUNK_ROWS
            pltpu.sync_copy(
                x_hbm.at[pl.ds(row0 * N_COLS, CHUNK_ROWS * N_COLS)], xbuf
            )

            def group_body(g, carry2):
                src_off = g * (4 * N_COLS)
                dst_off = g * (4 * N_SUB)
                for v in range(N_VECS):
                    iv = idx4_v[pl.ds(v * 16, 16)] + src_off
                    obuf[pl.ds(dst_off + v * 16, 16)] = plsc.load_gather(
                        xbuf, [iv]
                    )
                return carry2

            lax.fori_loop(0, GROUPS_PER_CHUNK, group_body, 0)
            pltpu.sync_copy(
                obuf, out_hbm.at[pl.ds(row0 * N_SUB, CHUNK_ROWS * N_SUB)]
            )
            return carry

        lax.fori_loop(0, N_CHUNKS, chunk_body, 0)

    return k(x_flat, idx)


@jax.jit
def kernel(x, sub_list_indices):
    out_flat = _sc_gather(x.reshape(-1), sub_list_indices.astype(jnp.int32))
    return out_flat.reshape(N_ROWS, N_SUB)


# speedup vs baseline: 1.1196x; 1.1196x over previous
import functools

import jax
import jax.numpy as jnp
from jax import lax
from jax.experimental import pallas as pl
from jax.experimental.pallas import tpu as pltpu
from jax.experimental.pallas import tpu_sc as plsc

N_ROWS = 16384
N_COLS = 1000
N_SUB = 100

NW = 32
ROWS_PER_W = N_ROWS // NW
CHUNK_ROWS = 32
N_CHUNKS = ROWS_PER_W // CHUNK_ROWS
GROUPS_PER_CHUNK = CHUNK_ROWS // 4
N_VECS = (4 * N_SUB) // 16
CHUNK_IN = CHUNK_ROWS * N_COLS
CHUNK_OUT = CHUNK_ROWS * N_SUB


def _sc_gather(x_flat, idx):
    mesh = plsc.VectorSubcoreMesh(core_axis_name="c", subcore_axis_name="s")

    @functools.partial(
        pl.kernel,
        mesh=mesh,
        out_type=jax.ShapeDtypeStruct((N_ROWS * N_SUB,), jnp.float32),
        compiler_params=pltpu.CompilerParams(needs_layout_passes=False),
        scratch_types=[
            pltpu.VMEM((N_SUB,), jnp.int32),
            pltpu.VMEM((CHUNK_IN,), jnp.float32),
            pltpu.VMEM((CHUNK_IN,), jnp.float32),
            pltpu.VMEM((ROWS_PER_W * N_SUB,), jnp.float32),
            pltpu.SemaphoreType.DMA,
            pltpu.SemaphoreType.DMA,
        ],
    )
    def k(x_hbm, idx_hbm, out_hbm, idx_v, xbuf0, xbuf1, obuf, sem0, sem1):
        wid = lax.axis_index("s") * 2 + lax.axis_index("c")
        base_row = wid * ROWS_PER_W

        pltpu.sync_copy(idx_hbm, idx_v)
        idx4 = []
        for v in range(N_VECS):
            kvec = lax.iota(jnp.int32, 16) + (v * 16)
            kmod = lax.rem(kvec, N_SUB)
            kdiv = lax.div(kvec, N_SUB)
            cols = plsc.load_gather(idx_v, [kmod])
            idx4.append(cols + kdiv * N_COLS)

        def start_in(chunk, buf, sem):
            src = x_hbm.at[pl.ds((base_row + chunk * CHUNK_ROWS) * N_COLS,
                                 CHUNK_IN)]
            pltpu.async_copy(src, buf, sem)

        def wait_in(buf, sem):
            pltpu.make_async_copy(x_hbm.at[pl.ds(0, CHUNK_IN)], buf, sem
                                  ).wait()

        def gather_chunk(chunk, buf):
            dst_base = chunk * CHUNK_OUT
            for g in range(GROUPS_PER_CHUNK):
                src_off = g * (4 * N_COLS)
                dst_off = dst_base + g * (4 * N_SUB)
                for v in range(N_VECS):
                    obuf[pl.ds(dst_off + v * 16, 16)] = plsc.load_gather(
                        buf, [idx4[v] + src_off]
                    )

        start_in(0, xbuf0, sem0)

        def pair_body(c, carry):
            start_in(2 * c + 1, xbuf1, sem1)
            wait_in(xbuf0, sem0)
            gather_chunk(2 * c, xbuf0)

            @pl.when(c < N_CHUNKS // 2 - 1)
            def _():
                start_in(2 * c + 2, xbuf0, sem0)

            wait_in(xbuf1, sem1)
            gather_chunk(2 * c + 1, xbuf1)
            return carry

        lax.fori_loop(0, N_CHUNKS // 2, pair_body, 0)
        pltpu.sync_copy(
            obuf, out_hbm.at[pl.ds(base_row * N_SUB, ROWS_PER_W * N_SUB)]
        )

    return k(x_flat, idx)


@jax.jit
def kernel(x, sub_list_indices):
    out_flat = _sc_gather(x.reshape(-1), sub_list_indices.astype(jnp.int32))
    return out_flat.reshape(N_ROWS, N_SUB)


# speedup vs baseline: 1.9336x; 1.7271x over previous
import functools

import jax
import jax.numpy as jnp
from jax import lax
from jax.experimental import pallas as pl
from jax.experimental.pallas import tpu as pltpu
from jax.experimental.pallas import tpu_sc as plsc

N_ROWS = 16384
N_COLS = 1000
N_SUB = 100

NW = 32
ROWS_PER_W = N_ROWS // NW
CHUNK_ROWS = 32
N_CHUNKS = ROWS_PER_W // CHUNK_ROWS
N_PAIRS = N_CHUNKS // 2
GROUPS_PER_CHUNK = CHUNK_ROWS // 4
N_VECS = (4 * N_SUB) // 16


def _sc_gather(x, idx):
    mesh = plsc.VectorSubcoreMesh(core_axis_name="c", subcore_axis_name="s")

    @functools.partial(
        pl.kernel,
        mesh=mesh,
        out_type=jax.ShapeDtypeStruct((N_ROWS, N_SUB), jnp.float32),
        compiler_params=pltpu.CompilerParams(needs_layout_passes=False),
        scratch_types=[
            pltpu.VMEM((N_SUB,), jnp.int32),
            pltpu.VMEM((CHUNK_ROWS, N_COLS), jnp.float32),
            pltpu.VMEM((CHUNK_ROWS, N_COLS), jnp.float32),
            pltpu.VMEM((CHUNK_ROWS, N_SUB), jnp.float32),
            pltpu.VMEM((CHUNK_ROWS, N_SUB), jnp.float32),
            pltpu.SemaphoreType.DMA,
            pltpu.SemaphoreType.DMA,
            pltpu.SemaphoreType.DMA,
            pltpu.SemaphoreType.DMA,
        ],
    )
    def k(x_hbm, idx_hbm, out_hbm, idx_v, xbuf0, xbuf1, obuf0, obuf1,
          si0, si1, so0, so1):
        wid = lax.axis_index("s") * 2 + lax.axis_index("c")
        base_row = wid * ROWS_PER_W

        pltpu.sync_copy(idx_hbm, idx_v)
        kdiv, kmod, cols = [], [], []
        for v in range(N_VECS):
            kvec = lax.iota(jnp.int32, 16) + (v * 16)
            kmod.append(lax.rem(kvec, N_SUB))
            kdiv.append(lax.div(kvec, N_SUB))
            cols.append(plsc.load_gather(idx_v, [kmod[v]]))

        def start_in(chunk, buf, sem):
            src = x_hbm.at[pl.ds(base_row + chunk * CHUNK_ROWS, CHUNK_ROWS)]
            pltpu.async_copy(src, buf, sem)

        def wait_in(buf, sem):
            pltpu.make_async_copy(
                x_hbm.at[pl.ds(0, CHUNK_ROWS)], buf, sem).wait()

        def start_out(chunk, buf, sem):
            dst = out_hbm.at[pl.ds(base_row + chunk * CHUNK_ROWS, CHUNK_ROWS)]
            pltpu.async_copy(buf, dst, sem)

        def wait_out(buf, sem):
            pltpu.make_async_copy(
                buf, out_hbm.at[pl.ds(0, CHUNK_ROWS)], sem).wait()

        def gather_chunk(xbuf, obuf):
            for g in range(GROUPS_PER_CHUNK):
                r0 = g * 4
                for v in range(N_VECS):
                    vals = plsc.load_gather(xbuf, [kdiv[v] + r0, cols[v]])
                    plsc.store_scatter(obuf, [kdiv[v] + r0, kmod[v]], vals)

        start_in(0, xbuf0, si0)

        def pair_body(c, carry):
            start_in(2 * c + 1, xbuf1, si1)
            wait_in(xbuf0, si0)

            @pl.when(c > 0)
            def _():
                wait_out(obuf0, so0)

            gather_chunk(xbuf0, obuf0)
            start_out(2 * c, obuf0, so0)

            @pl.when(c < N_PAIRS - 1)
            def _():
                start_in(2 * c + 2, xbuf0, si0)

            wait_in(xbuf1, si1)

            @pl.when(c > 0)
            def _():
                wait_out(obuf1, so1)

            gather_chunk(xbuf1, obuf1)
            start_out(2 * c + 1, obuf1, so1)
            return carry

        lax.fori_loop(0, N_PAIRS, pair_body, 0)
        wait_out(obuf0, so0)
        wait_out(obuf1, so1)

    return k(x, idx)


@jax.jit
def kernel(x, sub_list_indices):
    return _sc_gather(x, sub_list_indices.astype(jnp.int32))
